# baseline (speedup 1.0000x reference)
import jax
import jax.numpy as jnp
from jax import lax
from jax.experimental import pallas as pl
from jax.experimental.pallas import tpu as pltpu
from jax.experimental.pallas import tpu_sc as plsc

B = 1024
N = 100000
K = 64
NS = 16
CHUNK = 128
VPC = CHUNK // 16
NCH0 = 17
NCH1 = 15
E0 = NCH0 * CHUNK
E1 = NCH1 * CHUNK


def _gather_kernel(data_hbm, idx_hbm, out_hbm, idx_v, vals_v, sem):
    c_id = lax.axis_index("c")
    s_id = lax.axis_index("s")
    lane = lax.iota(jnp.int32, 16)

    def do_work(base, nch):
        per_w = nch * CHUNK
        pltpu.sync_copy(idx_hbm.at[pl.ds(base, per_w)], idx_v.at[pl.ds(0, per_w)])

        def chunk_body(ch, carry):
            b_hi = ((base + ch * CHUNK) >> 10) & 7
            for q in range(VPC):
                sbase = (b_hi << 10) + q * 16
                loc = ch * CHUNK + q * 16
                c = idx_v[pl.ds(loc, 16)]
                idx_v[pl.ds(loc, 16)] = (
                    ((c >> 3) << 13) + ((c & 7) << 7) + sbase + lane
                )
            pltpu.async_copy(
                data_hbm.at[idx_v.at[pl.ds(ch * CHUNK, CHUNK)]],
                vals_v.at[pl.ds(ch * CHUNK, CHUNK)],
                sem,
            )
            return carry

        lax.fori_loop(0, nch, chunk_body, 0)

        pltpu.make_async_copy(
            data_hbm.at[pl.ds(0, per_w)], vals_v.at[pl.ds(0, per_w)], sem
        ).wait()

        pltpu.sync_copy(vals_v.at[pl.ds(0, per_w)], out_hbm.at[pl.ds(base, per_w)])

    @pl.when(c_id == 0)
    def _():
        do_work(s_id * E0, NCH0)

    @pl.when(c_id != 0)
    def _():
        do_work(NS * E0 + s_id * E1, NCH1)


@jax.jit
def _gather_flat(data_flat, idx_flat):
    mesh = plsc.VectorSubcoreMesh(core_axis_name="c", subcore_axis_name="s")
    return pl.kernel(
        _gather_kernel,
        mesh=mesh,
        out_type=jax.ShapeDtypeStruct((B * K,), jnp.float32),
        scratch_types=[
            pltpu.VMEM((E0,), jnp.int32),
            pltpu.VMEM((E0,), jnp.float32),
            pltpu.SemaphoreType.DMA,
        ],
    )(data_flat, idx_flat)


def kernel(data, idx):
    data_flat = (
        data.T.reshape(N // 8, 8, B // 128, 128)
        .transpose(0, 2, 1, 3)
        .reshape(B * N)
    )
    idx_flat = (
        idx.astype(jnp.int32)
        .T.reshape(K // 8, 8, B // 128, 128)
        .transpose(0, 2, 1, 3)
        .reshape(B * K)
    )
    out_flat = _gather_flat(data_flat, idx_flat)
    return (
        out_flat.reshape(K // 8, B // 128, 8, 128)
        .transpose(0, 2, 1, 3)
        .reshape(K, B)
        .T
    )


# speedup vs baseline: 1.0273x; 1.0273x over previous
import jax
import jax.numpy as jnp
from jax import lax
from jax.experimental import pallas as pl
from jax.experimental.pallas import tpu as pltpu
from jax.experimental.pallas import tpu_sc as plsc

B = 1024
N = 100000
K = 64
NW = 32
PER_W = B * K // NW
CHUNK = 128
NCHUNK = PER_W // CHUNK
VPC = CHUNK // 16
HALF = PER_W // 2


def _gather_kernel(
    data_hbm, idx_hbm, out_hbm, idx_v, vals_v,
    sem_i, sem_o, sem_g0, sem_g1, sem_g2, sem_g3,
):
    w = lax.axis_index("s") * 2 + lax.axis_index("c")
    base = w * PER_W
    sem_g = [sem_g0, sem_g1, sem_g2, sem_g3]
    NQ = len(sem_g)
    QCH = NCHUNK // NQ
    QW = PER_W // NQ

    idx_loads = [
        pltpu.async_copy(
            idx_hbm.at[pl.ds(base + qq * QW, QW)],
            idx_v.at[pl.ds(qq * QW, QW)],
            sem_i,
        )
        for qq in range(NQ)
    ]

    lane = lax.iota(jnp.int32, 16)

    for qq in range(NQ):
        idx_loads[qq].wait()

        def chunk_body(ch, carry, _sem=sem_g[qq]):
            b_hi = ((base + ch * CHUNK) >> 10) & 7
            for q in range(VPC):
                sbase = (b_hi << 10) + q * 16
                loc = ch * CHUNK + q * 16
                c = idx_v[pl.ds(loc, 16)]
                idx_v[pl.ds(loc, 16)] = (
                    ((c >> 3) << 13) + ((c & 7) << 7) + sbase + lane
                )
            pltpu.async_copy(
                data_hbm.at[idx_v.at[pl.ds(ch * CHUNK, CHUNK)]],
                vals_v.at[pl.ds(ch * CHUNK, CHUNK)],
                _sem,
            )
            return carry

        lax.fori_loop(qq * QCH, (qq + 1) * QCH, chunk_body, 0)

    outs = []
    for qq in range(NQ):
        pltpu.make_async_copy(
            data_hbm.at[pl.ds(0, QW)],
            vals_v.at[pl.ds(qq * QW, QW)],
            sem_g[qq],
        ).wait()
        outs.append(
            pltpu.async_copy(
                vals_v.at[pl.ds(qq * QW, QW)],
                out_hbm.at[pl.ds(base + qq * QW, QW)],
                sem_o,
            )
        )
    for o in outs:
        o.wait()


@jax.jit
def _gather_flat(data_flat, idx_flat):
    mesh = plsc.VectorSubcoreMesh(core_axis_name="c", subcore_axis_name="s")
    return pl.kernel(
        _gather_kernel,
        mesh=mesh,
        out_type=jax.ShapeDtypeStruct((B * K,), jnp.float32),
        scratch_types=[
            pltpu.VMEM((PER_W,), jnp.int32),
            pltpu.VMEM((PER_W,), jnp.float32),
            pltpu.SemaphoreType.DMA,
            pltpu.SemaphoreType.DMA,
            pltpu.SemaphoreType.DMA,
            pltpu.SemaphoreType.DMA,
            pltpu.SemaphoreType.DMA,
            pltpu.SemaphoreType.DMA,
        ],
    )(data_flat, idx_flat)


def kernel(data, idx):
    data_flat = (
        data.T.reshape(N // 8, 8, B // 128, 128)
        .transpose(0, 2, 1, 3)
        .reshape(B * N)
    )
    idx_flat = (
        idx.astype(jnp.int32)
        .T.reshape(K // 8, 8, B // 128, 128)
        .transpose(0, 2, 1, 3)
        .reshape(B * K)
    )
    out_flat = _gather_flat(data_flat, idx_flat)
    return (
        out_flat.reshape(K // 8, B // 128, 8, 128)
        .transpose(0, 2, 1, 3)
        .reshape(K, B)
        .T
    )
